# baseline (speedup 1.0000x reference)
import jax
import jax.numpy as jnp
from jax.experimental import pallas as pl
from jax.experimental.pallas import tpu as pltpu

_B = 64
_L = 128
_DM = 768
_DF = 1536
_DL = 2048
_E = 8

_HW = _DM // 4
_HO = _DF // 4
_C1 = 128
_SPB = 4


def _moe_kernel(x_ref, dkp_ref, cn_ref, w1_ref, b1_ref, w2_ref, b2_ref,
                w3_ref, b3_ref, ewi_ref, ewo_ref, gwi_ref, gwo_ref,
                bi_ref, bo_ref, lg_ref, lb_ref, out_ref,
                wi_bf, wo_bf, stg_i, stg_o, stg_w1, gates_scr, ee_scr,
                ee_smem, sem_i, sem_o, sem_w1, sem_ee):
    s = pl.program_id(0)

    @pl.when(s == 0)
    def _prologue():
        wi_srcs = ([(ewi_ref, e, h, e) for e in range(_E) for h in range(4)]
                   + [(gwi_ref, 0, h, _E) for h in range(4)])
        wo_srcs = ([(ewo_ref, e, h, e) for e in range(_E) for h in range(4)]
                   + [(gwo_ref, 0, h, _E) for h in range(4)])

        def wi_cp(k):
            src, se, h, _ = wi_srcs[k]
            return pltpu.make_async_copy(
                src.at[se, pl.ds(h * _HW, _HW), :], stg_i.at[k % 2],
                sem_i.at[k % 2])

        def wo_cp(k):
            src, se, h, _ = wo_srcs[k]
            return pltpu.make_async_copy(
                src.at[se, pl.ds(h * _HO, _HO), :], stg_o.at[k % 2],
                sem_o.at[k % 2])

        def w1_cp(c):
            return pltpu.make_async_copy(
                w1_ref.at[pl.ds(c * _C1, _C1), :], stg_w1.at[c % 2],
                sem_w1.at[c % 2])

        wi_cp(0).start()
        wo_cp(0).start()
        w1_cp(0).start()
        w1_cp(1).start()

        dkp = dkp_ref[...]
        nc = _DL // _C1
        h_acc = jnp.zeros((_B, _DF), jnp.float32)
        for c in range(nc):
            w1_cp(c).wait()
            chunk = stg_w1[c % 2]
            h_acc = h_acc + jnp.dot(
                dkp[:, c * _C1:(c + 1) * _C1], chunk,
                preferred_element_type=jnp.float32)
            if c + 2 < nc:
                w1_cp(c + 2).start()
        h1 = jnp.maximum(h_acc + b1_ref[...], 0.0)
        h2 = jnp.maximum(cn_ref[...] * w2_ref[...] + b2_ref[...], 0.0)
        h = h1 + h2
        logits = (jnp.dot(h, w3_ref[...], preferred_element_type=jnp.float32)
                  + b3_ref[...])

        lane8 = jax.lax.broadcasted_iota(jnp.int32, (_B, _E), 1)
        m1 = jnp.max(logits, axis=1, keepdims=True)
        i1 = jnp.min(jnp.where(logits == m1, lane8, _E), axis=1,
                     keepdims=True)
        oh1 = lane8 == i1
        masked = jnp.where(oh1, -jnp.inf, logits)
        m2 = jnp.max(masked, axis=1, keepdims=True)
        i2 = jnp.min(jnp.where(masked == m2, lane8, _E), axis=1,
                     keepdims=True)
        oh2 = lane8 == i2
        sel = oh1 | oh2

        p = jnp.exp(logits - m1)
        p = p / jnp.sum(p, axis=1, keepdims=True)
        pm = jnp.where(sel, p, 0.0)
        gates_scr[...] = pm / (jnp.sum(pm, axis=1, keepdims=True) + 1e-9)
        ee_scr[...] = jnp.where(lane8 == 0, i1,
                                jnp.where(lane8 == 1, i2, 0))
        ee_dma = pltpu.make_async_copy(ee_scr, ee_smem, sem_ee.at[0])
        ee_dma.start()

        n = len(wi_srcs)
        for k in range(n):
            if k + 1 < n:
                wi_cp(k + 1).start()
                wo_cp(k + 1).start()
            wi_cp(k).wait()
            _, _, h_, de = wi_srcs[k]
            wi_bf[de, pl.ds(h_ * _HW, _HW), :] = stg_i[k % 2].astype(
                jnp.bfloat16)
            wo_cp(k).wait()
            _, _, h2_, de2 = wo_srcs[k]
            wo_bf[de2, pl.ds(h2_ * _HO, _HO), :] = stg_o[k % 2].astype(
                jnp.bfloat16)
        ee_dma.wait()

    def mlp_of(xv, xbv, e):
        h = jnp.maximum(
            jnp.dot(xbv, wi_bf[e], preferred_element_type=jnp.float32)
            + bi_ref[e], 0.0)
        o = (jnp.dot(h.astype(jnp.bfloat16), wo_bf[e],
                     preferred_element_type=jnp.float32)
             + bo_ref[e] + xv)
        mu = jnp.mean(o, axis=1, keepdims=True)
        var = jnp.mean((o - mu) ** 2, axis=1, keepdims=True)
        return (o - mu) / jnp.sqrt(var + 1e-5) * lg_ref[e] + lb_ref[e]

    xf = x_ref[...].reshape(_SPB * _L, _DM)
    xbf = xf.astype(jnp.bfloat16)
    gen = mlp_of(xf, xbf, _E)

    lane8 = jax.lax.broadcasted_iota(jnp.int32, (1, _E), 1)
    for i in range(_SPB):
        xi = xf[i * _L:(i + 1) * _L]
        xbi = xbf[i * _L:(i + 1) * _L]
        e0 = ee_smem[s * _SPB + i, 0]
        e1 = ee_smem[s * _SPB + i, 1]
        grow = gates_scr[pl.ds(s * _SPB + i, 1), :]
        w0 = jnp.sum(jnp.where(lane8 == e0, grow, 0.0))
        w1 = jnp.sum(jnp.where(lane8 == e1, grow, 0.0))
        tot = mlp_of(xi, xbi, e0) * w0 + mlp_of(xi, xbi, e1) * w1
        tot = tot.astype(jnp.bfloat16).astype(jnp.float32)
        out_ref[i] = gen[i * _L:(i + 1) * _L] + tot


def kernel(cycle_curve_data, cycle_numbers, DKP_embeddings, gate_W1, gate_b1,
           gate_W2, gate_b2, gate_W3, gate_b3, exp_Wi, exp_bi, exp_Wo, exp_bo,
           exp_g, exp_b, gen_Wi, gen_bi, gen_Wo, gen_bo, gen_g, gen_b):
    bi_all = jnp.concatenate([exp_bi, gen_bi[None]], axis=0)[:, None, :]
    bo_all = jnp.concatenate([exp_bo, gen_bo[None]], axis=0)[:, None, :]
    lg_all = jnp.concatenate([exp_g, gen_g[None]], axis=0)[:, None, :]
    lb_all = jnp.concatenate([exp_b, gen_b[None]], axis=0)[:, None, :]

    _c = lambda idx: pl.BlockSpec(memory_space=pl.ANY)
    final = pl.pallas_call(
        _moe_kernel,
        grid=(_B // _SPB,),
        in_specs=[
            pl.BlockSpec((_SPB, _L, _DM), lambda s: (s, 0, 0)),
            pl.BlockSpec((_B, _DL), lambda s: (0, 0)),
            pl.BlockSpec((_B, 1), lambda s: (0, 0)),
            pl.BlockSpec(memory_space=pl.ANY),
            pl.BlockSpec((1, _DF), lambda s: (0, 0)),
            pl.BlockSpec((1, _DF), lambda s: (0, 0)),
            pl.BlockSpec((1, _DF), lambda s: (0, 0)),
            pl.BlockSpec((_DF, _E), lambda s: (0, 0)),
            pl.BlockSpec((1, _E), lambda s: (0, 0)),
            pl.BlockSpec(memory_space=pl.ANY),
            pl.BlockSpec(memory_space=pl.ANY),
            pl.BlockSpec(memory_space=pl.ANY),
            pl.BlockSpec(memory_space=pl.ANY),
            pl.BlockSpec((_E + 1, 1, _DF), lambda s: (0, 0, 0)),
            pl.BlockSpec((_E + 1, 1, _DM), lambda s: (0, 0, 0)),
            pl.BlockSpec((_E + 1, 1, _DM), lambda s: (0, 0, 0)),
            pl.BlockSpec((_E + 1, 1, _DM), lambda s: (0, 0, 0)),
        ],
        out_specs=pl.BlockSpec((_SPB, _L, _DM), lambda s: (s, 0, 0)),
        scratch_shapes=[
            pltpu.VMEM((_E + 1, _DM, _DF), jnp.bfloat16),
            pltpu.VMEM((_E + 1, _DF, _DM), jnp.bfloat16),
            pltpu.VMEM((2, _HW, _DF), jnp.float32),
            pltpu.VMEM((2, _HO, _DM), jnp.float32),
            pltpu.VMEM((2, _C1, _DF), jnp.float32),
            pltpu.VMEM((_B, _E), jnp.float32),
            pltpu.VMEM((_B, _E), jnp.int32),
            pltpu.SMEM((_B, _E), jnp.int32),
            pltpu.SemaphoreType.DMA((2,)),
            pltpu.SemaphoreType.DMA((2,)),
            pltpu.SemaphoreType.DMA((2,)),
            pltpu.SemaphoreType.DMA((1,)),
        ],
        out_shape=jax.ShapeDtypeStruct((_B, _L, _DM), jnp.float32),
    )(cycle_curve_data, DKP_embeddings, cycle_numbers, gate_W1,
      gate_b1.reshape(1, _DF), gate_W2, gate_b2.reshape(1, _DF), gate_W3,
      gate_b3.reshape(1, _E), exp_Wi, exp_Wo,
      gen_Wi.reshape(1, _DM, _DF), gen_Wo.reshape(1, _DF, _DM),
      bi_all, bo_all, lg_all, lb_all)

    return (final, jnp.float32(0.0))
